# baseline (speedup 1.0000x reference)
import functools

import jax
import jax.numpy as jnp
from jax import lax
from jax.experimental import pallas as pl
from jax.experimental.pallas import tpu as pltpu
from jax.experimental.pallas import tpu_sc as plsc

_NUM_CORES = 2
_NUM_SUBCORES = 16
_NUM_WORKERS = _NUM_CORES * _NUM_SUBCORES


@functools.lru_cache(maxsize=None)
def _make_broadcast(batch, seq_len, feat):
    rows_per_worker = seq_len // _NUM_WORKERS
    tail_rows = seq_len - rows_per_worker * _NUM_WORKERS
    buf_rows = max(rows_per_worker, tail_rows, 1)
    mesh = plsc.VectorSubcoreMesh(core_axis_name="c", subcore_axis_name="s")

    @jax.jit
    @functools.partial(
        pl.kernel,
        mesh=mesh,
        out_type=jax.ShapeDtypeStruct((batch, seq_len, feat), jnp.float32),
        scratch_types=[
            pltpu.VMEM((buf_rows, feat), jnp.float32),
            pltpu.VMEM((buf_rows, feat), jnp.float32),
            pltpu.SemaphoreType.DMA,
            pltpu.SemaphoreType.DMA,
            pltpu.SemaphoreType.DMA,
        ],
    )
    def k(w_hbm, out_hbm, buf0, buf1, ls0, ls1, sem):
        wid = lax.axis_index("s") * _NUM_CORES + lax.axis_index("c")
        base = wid * rows_per_worker
        h = rows_per_worker // 2
        l0 = pltpu.async_copy(w_hbm.at[pl.ds(base, h)], buf0.at[pl.ds(0, h)], ls0)
        l1 = pltpu.async_copy(
            w_hbm.at[pl.ds(base + h, h)], buf1.at[pl.ds(0, h)], ls1
        )
        stores = []
        l0.wait()
        for b in range(batch):
            stores.append(
                pltpu.async_copy(
                    buf0.at[pl.ds(0, h)], out_hbm.at[b, pl.ds(base, h)], sem
                )
            )
        l1.wait()
        for b in range(batch):
            stores.append(
                pltpu.async_copy(
                    buf1.at[pl.ds(0, h)], out_hbm.at[b, pl.ds(base + h, h)], sem
                )
            )
        for s in stores:
            s.wait()

    return k


def kernel(seq_in_embeds, W):
    batch, seq_len = seq_in_embeds.shape[0], seq_in_embeds.shape[1]
    return _make_broadcast(batch, seq_len, W.shape[1])(W)


# speedup vs baseline: 1.0063x; 1.0063x over previous
import functools

import jax
import jax.numpy as jnp
from jax import lax
from jax.experimental import pallas as pl
from jax.experimental.pallas import tpu as pltpu
from jax.experimental.pallas import tpu_sc as plsc

_NUM_CORES = 2
_NUM_SUBCORES = 16
_NUM_WORKERS = _NUM_CORES * _NUM_SUBCORES


@functools.lru_cache(maxsize=None)
def _make_broadcast(batch, seq_len, feat):
    rows_per_worker = seq_len // _NUM_WORKERS
    tail_rows = seq_len - rows_per_worker * _NUM_WORKERS
    buf_rows = max(rows_per_worker, tail_rows, 1)
    mesh = plsc.VectorSubcoreMesh(core_axis_name="c", subcore_axis_name="s")

    @jax.jit
    @functools.partial(
        pl.kernel,
        mesh=mesh,
        out_type=jax.ShapeDtypeStruct((batch, seq_len, feat), jnp.float32),
        scratch_types=[
            pltpu.VMEM((buf_rows, feat), jnp.float32),
            pltpu.SemaphoreType.DMA,
        ],
    )
    def k(w_hbm, out_hbm, buf, sem):
        wid = lax.axis_index("s") * _NUM_CORES + lax.axis_index("c")

        def emit(base, nrows):
            pltpu.sync_copy(w_hbm.at[pl.ds(base, nrows)], buf.at[pl.ds(0, nrows)])
            stores = [
                pltpu.async_copy(
                    buf.at[pl.ds(0, nrows)],
                    out_hbm.at[b, pl.ds(base, nrows)],
                    sem,
                )
                for b in range(batch)
            ]
            for s in stores:
                s.wait()

        if rows_per_worker > 0:
            emit(wid * rows_per_worker, rows_per_worker)
        if tail_rows > 0:
            @pl.when(wid == 0)
            def _():
                emit(_NUM_WORKERS * rows_per_worker, tail_rows)

    return k


def kernel(seq_in_embeds, W):
    batch, seq_len = seq_in_embeds.shape[0], seq_in_embeds.shape[1]
    return _make_broadcast(batch, seq_len, W.shape[1])(W)
